# baseline (speedup 1.0000x reference)
import functools

import jax
import jax.numpy as jnp
from jax import lax
from jax.experimental import pallas as pl
from jax.experimental.pallas import tpu as pltpu
from jax.experimental.pallas import tpu_sc as plsc

BATCH = 16384
EMB = 32
VOCAB = 1000000
NC = 2
NS = 16
NW = NC * NS
LANES = 16
NWIN = (VOCAB + 127) // 128
WPT = (NWIN + NW - 1) // NW
BAND = 4
NWAVE = (WPT + BAND - 1) // BAND
SELCAP = 768
NGRP = BATCH // LANES
NSUB = 8
EMB_ROWS = BATCH + LANES

_mesh = plsc.VectorSubcoreMesh(core_axis_name="c", subcore_axis_name="s")


@functools.partial(
    pl.kernel,
    mesh=_mesh,
    compiler_params=pltpu.CompilerParams(needs_layout_passes=False),
    out_type=(
        jax.ShapeDtypeStruct((EMB_ROWS, 128), jnp.float32),
        jax.ShapeDtypeStruct((EMB_ROWS, 128), jnp.float32),
    ),
    scratch_types=[
        pltpu.VMEM((BATCH,), jnp.int32),
        pltpu.VMEM((BATCH,), jnp.int32),
        pltpu.VMEM((2, SELCAP), jnp.int32),
        pltpu.VMEM((2, SELCAP), jnp.int32),
        pltpu.VMEM((2, SELCAP), jnp.int32),
        pltpu.VMEM((2, 2, EMB, BAND * 128), jnp.float32),
        pltpu.VMEM((NSUB, LANES, 128), jnp.float32),
        pltpu.VMEM((NSUB, 1, LANES), jnp.int32),
        pltpu.SemaphoreType.DMA((2, 2)),
        pltpu.SemaphoreType.DMA((NSUB,)),
    ],
)
def _gather_phase(first_hbm, second_hbm, w1t_hbm, w2t_hbm,
                  emb1_hbm, emb2_hbm,
                  idx1_v, idx2_v, selw_v, sell_v, selb_v,
                  ring_v, rows_v, idx3_v, bsem, ssem):
    wid = lax.axis_index("s") * NC + lax.axis_index("c")
    lo = wid * WPT
    hi = lax.min(lo + WPT, NWIN)
    iota = lax.iota(jnp.int32, LANES)
    emb_hbm = (emb1_hbm, emb2_hbm)

    def wstart(q):
        return lax.min(lo + q * BAND, NWIN - BAND)

    def issue_band(q, sp):
        off = pl.multiple_of(wstart(q) * 128, 128)
        pltpu.async_copy(w1t_hbm.at[:, pl.ds(off, BAND * 128)],
                         ring_v.at[sp, 0], bsem.at[sp, 0])
        pltpu.async_copy(w2t_hbm.at[:, pl.ds(off, BAND * 128)],
                         ring_v.at[sp, 1], bsem.at[sp, 1])

    def drain_band(sp):
        for t in range(2):
            pltpu.make_async_copy(w1t_hbm.at[:, pl.ds(0, BAND * 128)],
                                  ring_v.at[sp, t], bsem.at[sp, t]).wait()

    def drain_scat(sub):
        pltpu.make_async_copy(rows_v.at[sub],
                              emb1_hbm.at[idx3_v.at[sub, 0]],
                              ssem.at[sub]).wait()

    issue_band(0, 0)
    issue_band(1, 1)

    pltpu.sync_copy(first_hbm, idx1_v)
    pltpu.sync_copy(second_hbm, idx2_v)

    def scan_body(g, carry):
        cnts = list(carry)
        for t, idx_v in ((0, idx1_v), (1, idx2_v)):
            r = idx_v[pl.ds(g * LANES, LANES)]
            w = r >> 7
            m = jnp.logical_and(w >= lo, w < hi)
            mi = m.astype(jnp.int32)
            pos = jnp.full((LANES,), cnts[t], jnp.int32) + lax.cumsum(mi) - mi
            plsc.store_scatter(selw_v, [jnp.full((LANES,), t, jnp.int32),
                                        pos], w, mask=m)
            plsc.store_scatter(sell_v, [jnp.full((LANES,), t, jnp.int32),
                                        pos],
                               lax.bitwise_and(r, jnp.int32(127)), mask=m)
            plsc.store_scatter(selb_v, [jnp.full((LANES,), t, jnp.int32),
                                        pos], g * LANES + iota, mask=m)
            cnts[t] = cnts[t] + plsc.all_reduce_population_count(m)[0]
        return tuple(cnts)

    cnt1, cnt2 = lax.fori_loop(0, NGRP, scan_body,
                               (jnp.int32(0), jnp.int32(0)))
    cnts = (cnt1, cnt2)

    def extract_table(t, q, sp, vcnt0):
        ws = wstart(q)
        cnt = cnts[t]
        ngroups = (cnt + LANES - 1) // LANES

        def grp_body(gg, vcnt):
            tsplat = jnp.full((LANES,), t, jnp.int32)
            wv = selw_v[t, pl.ds(gg * LANES, LANES)]
            valid = (gg * LANES + iota) < jnp.full((LANES,), cnt, jnp.int32)
            inband = jnp.logical_and(
                valid,
                jnp.logical_and(wv >= jnp.full((LANES,), ws, jnp.int32),
                                wv < jnp.full((LANES,), ws + BAND,
                                              jnp.int32)))
            npop = plsc.all_reduce_population_count(inband)[0]

            @pl.when(npop > 0)
            def _():
                sub = lax.bitwise_and(vcnt, jnp.int32(NSUB - 1))

                @pl.when(vcnt >= NSUB)
                def _():
                    drain_scat(sub)

                lv = sell_v[t, pl.ds(gg * LANES, LANES)]
                bv = selb_v[t, pl.ds(gg * LANES, LANES)]
                slot = lax.max(jnp.zeros((LANES,), jnp.int32),
                               lax.min(wv - ws,
                                       jnp.full((LANES,), BAND - 1,
                                                jnp.int32)))
                svec = slot * 128 + lax.bitwise_and(lv, jnp.int32(127))
                spv = jnp.full((LANES,), sp, jnp.int32)
                subv = jnp.full((LANES,), sub, jnp.int32)
                for c in range(EMB):
                    cv = jnp.full((LANES,), c, jnp.int32)
                    v = plsc.load_gather(ring_v, [spv, tsplat, cv, svec])
                    plsc.store_scatter(rows_v, [subv, iota, cv], v)
                dstb = lax.select(inband, bv, BATCH + iota)
                idx3_v[sub, 0, :] = dstb
                pltpu.async_copy(rows_v.at[sub],
                                 emb_hbm[t].at[idx3_v.at[sub, 0]],
                                 ssem.at[sub])

            return vcnt + lax.select(npop > 0, jnp.int32(1), jnp.int32(0))

        return lax.fori_loop(0, ngroups, grp_body, vcnt0)

    def wave_body(i, vcnt):
        for sp in range(2):
            q = i * 2 + sp
            drain_band(sp)
            vcnt = extract_table(0, q, sp, vcnt)
            vcnt = extract_table(1, q, sp, vcnt)

            @pl.when(q + 2 < NWAVE)
            def _():
                issue_band(q + 2, sp)
        return vcnt

    vcnt = lax.fori_loop(0, NWAVE // 2, wave_body, jnp.int32(0))

    for k in range(NSUB):
        @pl.when(vcnt > k)
        def _():
            drain_scat(lax.bitwise_and(vcnt - 1 - k, jnp.int32(NSUB - 1)))


@functools.partial(
    pl.kernel,
    mesh=_mesh,
    compiler_params=pltpu.CompilerParams(needs_layout_passes=False),
    out_type=jax.ShapeDtypeStruct((4, BATCH // 128, 8, 128), jnp.float32),
    scratch_types=[
        pltpu.VMEM((128, 128), jnp.float32),
        pltpu.VMEM((128, 128), jnp.float32),
        pltpu.VMEM((4, BATCH // 128 // NW, 8, 128), jnp.float32),
        pltpu.SemaphoreType.DMA,
        pltpu.SemaphoreType.DMA,
    ],
)
def _join_phase(emb1_hbm, emb2_hbm, out_hbm, e1_v, e2_v, ost_v, sem1, sem2):
    wid = lax.axis_index("s") * NC + lax.axis_index("c")
    bpw = BATCH // NW
    tbw = bpw // 128
    base = wid * bpw
    iota = lax.iota(jnp.int32, LANES)
    chi_lo = iota >> 3
    clo_lo = lax.bitwise_and(iota, jnp.int32(7))
    chi_hi = (iota + LANES) >> 3

    def chunk_body(k, _):
        cp1 = pltpu.async_copy(emb1_hbm.at[pl.ds(base + k * 128, 128)],
                               e1_v, sem1)
        cp2 = pltpu.async_copy(emb2_hbm.at[pl.ds(base + k * 128, 128)],
                               e2_v, sem2)
        cp1.wait()
        cp2.wait()
        for grp in range(128 // LANES):
            bl = grp * LANES + iota
            for c in range(EMB):
                cv = jnp.full((LANES,), c, jnp.int32)
                v1 = plsc.load_gather(e1_v, [bl, cv])
                v2 = plsc.load_gather(e2_v, [bl, cv])
                ost_v[c >> 3, k, c & 7,
                      pl.ds(grp * LANES, LANES)] = v1 * v2
        return ()

    lax.fori_loop(0, tbw, chunk_body, ())

    for c_hi in range(4):
        pltpu.sync_copy(ost_v.at[c_hi],
                        out_hbm.at[c_hi, pl.ds(wid * tbw, tbw)])


def kernel(first, second, W1, W2):
    emb1, emb2 = _gather_phase(first, second, W1.T, W2.T)
    out4 = _join_phase(emb1, emb2)
    return out4.transpose(0, 2, 1, 3).reshape(EMB, BATCH).T


# speedup vs baseline: 4.3678x; 4.3678x over previous
import functools

import jax
import jax.numpy as jnp
from jax import lax
from jax.experimental import pallas as pl
from jax.experimental.pallas import tpu as pltpu
from jax.experimental.pallas import tpu_sc as plsc

BATCH = 16384
EMB = 32
NC = 2
NS = 16
NW = NC * NS
BPW = BATCH // NW
LANES = 16
NBUF = 8
NGRP = BPW // LANES
TB_PER_W = BPW // 128

_mesh = plsc.VectorSubcoreMesh(core_axis_name="c", subcore_axis_name="s")

@functools.partial(
    pl.kernel,
    mesh=_mesh,
    compiler_params=pltpu.CompilerParams(needs_layout_passes=False),
    out_type=jax.ShapeDtypeStruct((4, BATCH // 128, 8, 128), jnp.float32),
    scratch_types=[
        pltpu.VMEM((BPW,), jnp.int32),
        pltpu.VMEM((BPW,), jnp.int32),
        pltpu.VMEM((NBUF, EMB, 128), jnp.float32),
        pltpu.VMEM((NBUF, EMB, 128), jnp.float32),
        pltpu.VMEM((4, TB_PER_W, 8, 128), jnp.float32),
        pltpu.SemaphoreType.DMA((2, NBUF)),
    ],
)
def _interac(first_hbm, second_hbm, w1t_hbm, w2t_hbm, out_hbm,
             idx1_v, idx2_v, g1_v, g2_v, ost_v, sems):
    wid = lax.axis_index("s") * NC + lax.axis_index("c")
    base = wid * BPW

    pltpu.sync_copy(first_hbm.at[pl.ds(base, BPW)], idx1_v)
    pltpu.sync_copy(second_hbm.at[pl.ds(base, BPW)], idx2_v)

    iota = lax.iota(jnp.int32, LANES)
    chi_lo = iota >> 3
    clo_lo = lax.bitwise_and(iota, jnp.int32(7))
    chi_hi = (iota + LANES) >> 3
    clo_hi = clo_lo

    def issue(gv1, gv2, j):
        s = j % NBUF
        w1 = pl.multiple_of(gv1[j], 128)
        w2 = pl.multiple_of(gv2[j], 128)
        pltpu.async_copy(w1t_hbm.at[:, pl.ds(w1, 128)], g1_v.at[s],
                         sems.at[0, s])
        pltpu.async_copy(w2t_hbm.at[:, pl.ds(w2, 128)], g2_v.at[s],
                         sems.at[1, s])

    def drain(s):
        pltpu.make_async_copy(w1t_hbm.at[:, pl.ds(0, 128)], g1_v.at[s],
                              sems.at[0, s]).wait()
        pltpu.make_async_copy(w2t_hbm.at[:, pl.ds(0, 128)], g2_v.at[s],
                              sems.at[1, s]).wait()

    def extract(b_prev, lv1, lv2, jlane, s):
        l1 = jnp.full((LANES,), lv1[jlane], jnp.int32)
        l2 = jnp.full((LANES,), lv2[jlane], jnp.int32)
        sv = jnp.full((LANES,), s, jnp.int32)
        v1a = plsc.load_gather(g1_v, [sv, iota, l1])
        v1b = plsc.load_gather(g1_v, [sv, iota + LANES, l1])
        v2a = plsc.load_gather(g2_v, [sv, iota, l2])
        v2b = plsc.load_gather(g2_v, [sv, iota + LANES, l2])
        tb = jnp.full((LANES,), b_prev >> 7, jnp.int32)
        blo = jnp.full((LANES,), b_prev & 127, jnp.int32)
        plsc.store_scatter(ost_v, [chi_lo, tb, clo_lo, blo], v1a * v2a)
        plsc.store_scatter(ost_v, [chi_hi, tb, clo_hi, blo], v1b * v2b)

    def body(g, carry):
        plv1, plv2 = carry
        i1 = idx1_v[pl.ds(g * LANES, LANES)]
        i2 = idx2_v[pl.ds(g * LANES, LANES)]
        gv1 = (i1 >> 7) * 128
        gv2 = (i2 >> 7) * 128
        lv1 = lax.bitwise_and(i1, jnp.int32(127))
        lv2 = lax.bitwise_and(i2, jnp.int32(127))
        for j in range(LANES):
            s = j % NBUF
            if j < NBUF:
                @pl.when(g > 0)
                def _():
                    drain(s)
                    extract(g * LANES + j - NBUF, plv1, plv2, j + NBUF, s)
            else:
                drain(s)
                extract(g * LANES + j - NBUF, lv1, lv2, j - NBUF, s)
            issue(gv1, gv2, j)
        return (lv1, lv2)

    zeros = jnp.zeros((LANES,), jnp.int32)
    lv1, lv2 = lax.fori_loop(0, NGRP, body, (zeros, zeros))

    for j in range(NBUF):
        drain(j)
        extract(BPW - NBUF + j, lv1, lv2, NBUF + j, j)

    for c_hi in range(4):
        pltpu.sync_copy(ost_v.at[c_hi],
                        out_hbm.at[c_hi, pl.ds(wid * TB_PER_W, TB_PER_W)])


def kernel(first, second, W1, W2):
    out4 = _interac(first, second, W1.T, W2.T)
    return out4.transpose(0, 2, 1, 3).reshape(EMB, BATCH).T
